# baseline (speedup 1.0000x reference)
import dataclasses

import jax
import jax.numpy as jnp
from jax import lax
from jax.experimental import pallas as pl
from jax.experimental.pallas import tpu as pltpu
from jax.experimental.pallas import tpu_sc as plsc

_B = 16384
_D = 64
_L = 16
_NC = 2
_NS = 16
_NW = _NC * _NS
_BPW = _B // _NW
_W = 128
_NWIN = _BPW // _W
_RING = 3
_WAVE = _L
_NWAVES = _BPW // _WAVE


def _sc_body(uid_hbm, iid_hbm, uf_hbm, if_hbm, ub_hbm,
             ib_hbm, gb_hbm, out_hbm, uid_v, iid_v,
             ublk_v, iblk_v, ub_v, ib_v, gb_v, out_v, bsem, fsem):
  wid = lax.axis_index("s") * _NC + lax.axis_index("c")
  base = wid * _BPW
  pltpu.sync_copy(uid_hbm.at[pl.ds(base, _BPW)], uid_v)
  pltpu.sync_copy(iid_hbm.at[pl.ds(base, _BPW)], iid_v)
  pltpu.sync_copy(gb_hbm, gb_v)

  bias_copies = []
  for j in range(_NWIN):
    sl = pl.ds(j * _W, _W)
    bias_copies.append(
        pltpu.async_copy(ub_hbm.at[uid_v.at[sl]], ub_v.at[sl], bsem))
    bias_copies.append(
        pltpu.async_copy(ib_hbm.at[iid_v.at[sl]], ib_v.at[sl], bsem))

  def fire(g):
    ring = g % _RING
    uvec = uid_v[pl.ds(g * _WAVE, _L)]
    ivec = iid_v[pl.ds(g * _WAVE, _L)]
    ub8 = (uvec >> 3) << 3
    ib8 = (ivec >> 3) << 3
    for r in range(_WAVE):
      u0 = pl.multiple_of(ub8[r], 8)
      i0 = pl.multiple_of(ib8[r], 8)
      pltpu.async_copy(uf_hbm.at[pl.ds(u0, 8), :],
                       ublk_v.at[ring, r], fsem.at[ring])
      pltpu.async_copy(if_hbm.at[pl.ds(i0, 8), :],
                       iblk_v.at[ring, r], fsem.at[ring])

  def drain(g):
    ring = g % _RING
    for r in range(_WAVE):
      pltpu.make_async_copy(uf_hbm.at[pl.ds(0, 8), :],
                            ublk_v.at[ring, r], fsem.at[ring]).wait()
      pltpu.make_async_copy(if_hbm.at[pl.ds(0, 8), :],
                            iblk_v.at[ring, r], fsem.at[ring]).wait()

  lane = lax.iota(jnp.int32, _L)
  for g in range(_RING - 1):
    fire(g)

  @pl.loop(0, _NWAVES)
  def _(g):
    @pl.when(g + _RING - 1 < _NWAVES)
    def _():
      fire(g + _RING - 1)

    drain(g)
    ring = g % _RING
    b0 = g * _WAVE
    urem = uid_v[pl.ds(b0, _L)] & 7
    irem = iid_v[pl.ds(b0, _L)] & 7
    acc = jnp.zeros((_L,), jnp.float32)
    for r in range(_WAVE):
      ur = urem[r]
      ir = irem[r]
      p = (ublk_v[ring, r, ur, pl.ds(0, _L)] *
           iblk_v[ring, r, ir, pl.ds(0, _L)])
      for k in range(1, _D // _L):
        p += (ublk_v[ring, r, ur, pl.ds(k * _L, _L)] *
              iblk_v[ring, r, ir, pl.ds(k * _L, _L)])
      acc = jnp.where(lane == r, acc + jnp.sum(p), acc)
    out_v[pl.ds(b0, _L)] = acc

  for c in bias_copies:
    c.wait()

  gb16 = gb_v[...]

  @pl.loop(0, _BPW // _L)
  def _(g):
    sl = pl.ds(g * _L, _L)
    out_v[sl] = out_v[sl] + ub_v[sl] + ib_v[sl] + gb16

  pltpu.sync_copy(out_v, out_hbm.at[pl.ds(base, _BPW)])


def kernel(user_ids, item_ids, user_factors, item_factors, user_biases,
           item_biases, global_bias):
  mesh = plsc.VectorSubcoreMesh(core_axis_name="c", subcore_axis_name="s")
  cp = pltpu.CompilerParams(use_tc_tiling_on_sc=True)
  if "needs_layout_passes" in pltpu.CompilerParams.__dataclass_fields__:
    cp = dataclasses.replace(cp, needs_layout_passes=False)
  sc_call = pl.kernel(
      _sc_body,
      mesh=mesh,
      compiler_params=cp,
      out_type=jax.ShapeDtypeStruct((_B,), jnp.float32),
      scratch_types=[
          pltpu.VMEM((_BPW,), jnp.int32),
          pltpu.VMEM((_BPW,), jnp.int32),
          pltpu.VMEM((_RING, _WAVE, 8, _D), jnp.float32),
          pltpu.VMEM((_RING, _WAVE, 8, _D), jnp.float32),
          pltpu.VMEM((_BPW,), jnp.float32),
          pltpu.VMEM((_BPW,), jnp.float32),
          pltpu.VMEM((_L,), jnp.float32),
          pltpu.VMEM((_BPW,), jnp.float32),
          pltpu.SemaphoreType.DMA,
          pltpu.SemaphoreType.DMA((_RING,)),
      ],
  )
  out = sc_call(
      user_ids,
      item_ids,
      user_factors,
      item_factors,
      lax.squeeze(user_biases, (1,)),
      lax.squeeze(item_biases, (1,)),
      jnp.broadcast_to(global_bias, (_L,)),
  )
  ug = jnp.take(user_factors, user_ids, axis=0)[:, 0]
  ig = jnp.take(item_factors, item_ids, axis=0)[:, 0]
  zero = jnp.where(jnp.abs(ug) + jnp.abs(ig) > 1e30, ug, 0.0)
  return out + zero


# speedup vs baseline: 1.1021x; 1.1021x over previous
import dataclasses

import jax
import jax.numpy as jnp
from jax import lax
from jax.experimental import pallas as pl
from jax.experimental.pallas import tpu as pltpu
from jax.experimental.pallas import tpu_sc as plsc

_B = 16384
_D = 64
_L = 16
_NC = 2
_NS = 16
_NW = _NC * _NS
_BPW = _B // _NW
_W = 128
_NWIN = _BPW // _W
_RING = 3
_WAVE = _L
_NWAVES = _BPW // _WAVE


def _sc_body(uid_hbm, iid_hbm, uf_hbm, if_hbm, ub_hbm,
             ib_hbm, gb_hbm, out_hbm, uid_v, iid_v,
             ublk_v, iblk_v, ub_v, ib_v, gb_v, out_v, bsem, fsem):
  wid = lax.axis_index("s") * _NC + lax.axis_index("c")
  base = wid * _BPW
  pltpu.sync_copy(uid_hbm.at[pl.ds(base, _BPW)], uid_v)
  pltpu.sync_copy(iid_hbm.at[pl.ds(base, _BPW)], iid_v)
  pltpu.sync_copy(gb_hbm, gb_v)

  bias_copies = []
  for j in range(_NWIN):
    sl = pl.ds(j * _W, _W)
    bias_copies.append(
        pltpu.async_copy(ub_hbm.at[uid_v.at[sl]], ub_v.at[sl], bsem))
    bias_copies.append(
        pltpu.async_copy(ib_hbm.at[iid_v.at[sl]], ib_v.at[sl], bsem))

  def fire(g):
    ring = g % _RING
    uvec = uid_v[pl.ds(g * _WAVE, _L)]
    ivec = iid_v[pl.ds(g * _WAVE, _L)]
    ub8 = (uvec >> 3) << 3
    ib8 = (ivec >> 3) << 3
    for r in range(_WAVE):
      u0 = pl.multiple_of(ub8[r], 8)
      i0 = pl.multiple_of(ib8[r], 8)
      pltpu.async_copy(uf_hbm.at[pl.ds(u0, 8), :],
                       ublk_v.at[ring, r], fsem.at[ring])
      pltpu.async_copy(if_hbm.at[pl.ds(i0, 8), :],
                       iblk_v.at[ring, r], fsem.at[ring])

  def drain(g):
    ring = g % _RING
    for r in range(_WAVE):
      pltpu.make_async_copy(uf_hbm.at[pl.ds(0, 8), :],
                            ublk_v.at[ring, r], fsem.at[ring]).wait()
      pltpu.make_async_copy(if_hbm.at[pl.ds(0, 8), :],
                            iblk_v.at[ring, r], fsem.at[ring]).wait()

  lane = lax.iota(jnp.int32, _L)
  for g in range(_RING - 1):
    fire(g)

  @pl.loop(0, _NWAVES)
  def _(g):
    @pl.when(g + _RING - 1 < _NWAVES)
    def _():
      fire(g + _RING - 1)

    drain(g)
    ring = g % _RING
    b0 = g * _WAVE
    urem = uid_v[pl.ds(b0, _L)] & 7
    irem = iid_v[pl.ds(b0, _L)] & 7
    acc = jnp.zeros((_L,), jnp.float32)
    for r in range(_WAVE):
      ur = urem[r]
      ir = irem[r]
      p = (ublk_v[ring, r, ur, pl.ds(0, _L)] *
           iblk_v[ring, r, ir, pl.ds(0, _L)])
      for k in range(1, _D // _L):
        p += (ublk_v[ring, r, ur, pl.ds(k * _L, _L)] *
              iblk_v[ring, r, ir, pl.ds(k * _L, _L)])
      acc = jnp.where(lane == r, acc + jnp.sum(p), acc)
    out_v[pl.ds(b0, _L)] = acc

  for c in bias_copies:
    c.wait()

  gb16 = gb_v[...]

  @pl.loop(0, _BPW // _L)
  def _(g):
    sl = pl.ds(g * _L, _L)
    out_v[sl] = out_v[sl] + ub_v[sl] + ib_v[sl] + gb16

  pltpu.sync_copy(out_v, out_hbm.at[pl.ds(base, _BPW)])


_CHB = 65536


def _bias_1d(bias_t):
  n = bias_t.shape[1]

  def body(x_ref, o_ref):
    o_ref[...] = x_ref[...].reshape(_CHB)

  return pl.pallas_call(
      body,
      grid=(pl.cdiv(n, _CHB),),
      in_specs=[pl.BlockSpec((1, _CHB), lambda i: (0, i))],
      out_specs=pl.BlockSpec((_CHB,), lambda i: (i,)),
      out_shape=jax.ShapeDtypeStruct((n,), jnp.float32),
  )(bias_t)


def kernel(user_ids, item_ids, user_factors, item_factors, user_biases,
           item_biases, global_bias):
  mesh = plsc.VectorSubcoreMesh(core_axis_name="c", subcore_axis_name="s")
  cp = pltpu.CompilerParams(use_tc_tiling_on_sc=True)
  if "needs_layout_passes" in pltpu.CompilerParams.__dataclass_fields__:
    cp = dataclasses.replace(cp, needs_layout_passes=False)
  sc_call = pl.kernel(
      _sc_body,
      mesh=mesh,
      compiler_params=cp,
      out_type=jax.ShapeDtypeStruct((_B,), jnp.float32),
      scratch_types=[
          pltpu.VMEM((_BPW,), jnp.int32),
          pltpu.VMEM((_BPW,), jnp.int32),
          pltpu.VMEM((_RING, _WAVE, 8, _D), jnp.float32),
          pltpu.VMEM((_RING, _WAVE, 8, _D), jnp.float32),
          pltpu.VMEM((_BPW,), jnp.float32),
          pltpu.VMEM((_BPW,), jnp.float32),
          pltpu.VMEM((_L,), jnp.float32),
          pltpu.VMEM((_BPW,), jnp.float32),
          pltpu.SemaphoreType.DMA,
          pltpu.SemaphoreType.DMA((_RING,)),
      ],
  )
  return sc_call(
      user_ids,
      item_ids,
      user_factors,
      item_factors,
      _bias_1d(user_biases.T),
      _bias_1d(item_biases.T),
      jnp.broadcast_to(global_bias, (_L,)),
  )
